# Initial kernel scaffold; baseline (speedup 1.0000x reference)
import jax
import jax.numpy as jnp
from jax.experimental import pallas as pl


def kernel(x, edge_index, batch, W_in, b_in, eps, W1, b1, g1, be1, W2, b2, g2, be2, Wh1, bh1, Wh2, bh2):
    raise NotImplementedError("write your pallas kernel here")


# speedup vs baseline: 9.3146x; 9.3146x over previous
import functools

import jax
import jax.numpy as jnp
from jax import lax
from jax.experimental import pallas as pl
from jax.experimental.pallas import tpu as pltpu
from jax.experimental.pallas import tpu_sc as plsc

_N = 10000
_E = 320000
_D = 128
_L = 5
_G = 256

_NC = 2
_NS = 16
_NW = _NC * _NS

_CH = 125
_TPW = _E // _NW
_NCHUNK = _TPW // _CH
_NPAIR = _NCHUNK // 2
_DHALF = _NCHUNK // 2
_STRIPE = 624
_REM = _N - _NS * _STRIPE


_sc_mesh = plsc.VectorSubcoreMesh(core_axis_name="c", subcore_axis_name="s")


@functools.partial(
    pl.kernel,
    out_type=jax.ShapeDtypeStruct((_NC, _N, _D), jnp.float32),
    mesh=_sc_mesh,
    scratch_types=[
        pltpu.VMEM((_NCHUNK, _CH), jnp.int32),
        pltpu.VMEM((_DHALF, _CH), jnp.int32),
        pltpu.VMEM((2, _CH, _D), jnp.float32),
        pltpu.VMEM_SHARED((_N, _D), jnp.float32),
        pltpu.SemaphoreType.DMA,
        pltpu.SemaphoreType.DMA,
    ],
)
def _sc_gather_scatter(h_hbm, src_hbm, dst_hbm, out_hbm,
                       src_v, dst_v, rows_v, agg_sh, sem0, sem1):
    c = lax.axis_index("c")
    s = lax.axis_index("s")
    wid = c * _NS + s

    row0 = wid * _NCHUNK
    pltpu.sync_copy(src_hbm.at[pl.ds(row0, _NCHUNK)], src_v)
    pltpu.sync_copy(dst_hbm.at[pl.ds(row0, _DHALF)], dst_v)

    zv = jnp.zeros((16,), jnp.float32)

    def _zero_row(i, carry):
        for j in range(_D // 16):
            rows_v[0, i, pl.ds(j * 16, 16)] = zv
        return carry

    lax.fori_loop(0, _CH, _zero_row, 0)
    for k in range(5):
        pltpu.sync_copy(rows_v.at[0, pl.ds(0, 120)],
                        agg_sh.at[pl.ds(s * _STRIPE + k * 120, 120)])
    pltpu.sync_copy(rows_v.at[0, pl.ds(0, 24)],
                    agg_sh.at[pl.ds(s * _STRIPE + 600, 24)])

    @pl.when(s == 0)
    def _zero_rem():
        pltpu.sync_copy(rows_v.at[0, pl.ds(0, _REM)],
                        agg_sh.at[pl.ds(_NS * _STRIPE, _REM)])

    plsc.subcore_barrier()

    pltpu.async_copy(h_hbm.at[src_v.at[0]], rows_v.at[0], sem0)

    def _pair(i, carry):
        j0 = 2 * i

        @pl.when(j0 == _DHALF)
        def _reload_dst():
            pltpu.sync_copy(dst_hbm.at[pl.ds(row0 + _DHALF, _DHALF)], dst_v)

        r0 = lax.rem(j0, _DHALF)
        pltpu.make_async_copy(h_hbm.at[src_v.at[j0]], rows_v.at[0], sem0).wait()
        pltpu.async_copy(h_hbm.at[src_v.at[j0 + 1]], rows_v.at[1], sem1)
        pltpu.sync_copy(rows_v.at[0], agg_sh.at[dst_v.at[r0]], add=True)
        pltpu.make_async_copy(h_hbm.at[src_v.at[j0 + 1]], rows_v.at[1], sem1).wait()

        @pl.when(i + 1 < _NPAIR)
        def _prefetch():
            pltpu.async_copy(h_hbm.at[src_v.at[j0 + 2]], rows_v.at[0], sem0)

        pltpu.sync_copy(rows_v.at[1], agg_sh.at[dst_v.at[r0 + 1]], add=True)
        return carry

    lax.fori_loop(0, _NPAIR, _pair, 0)
    plsc.subcore_barrier()

    pltpu.sync_copy(agg_sh.at[pl.ds(s * _STRIPE, _STRIPE)],
                    out_hbm.at[c, pl.ds(s * _STRIPE, _STRIPE)])

    @pl.when(s == 0)
    def _copy_rem():
        pltpu.sync_copy(agg_sh.at[pl.ds(_NS * _STRIPE, _REM)],
                        out_hbm.at[c, pl.ds(_NS * _STRIPE, _REM)])


def _bn_relu(t, g, be):
    m = jnp.mean(t, axis=0, keepdims=True)
    v = jnp.mean((t - m) ** 2, axis=0, keepdims=True)
    return jnp.maximum(g * (t - m) * lax.rsqrt(v + 1e-5) + be, 0.0)


def _mlp_in_body(x_ref, W_ref, b_ref, o_ref):
    o_ref[...] = jnp.maximum(
        jnp.dot(x_ref[...], W_ref[...], preferred_element_type=jnp.float32)
        + b_ref[...], 0.0)


_mlp_in = pl.pallas_call(
    _mlp_in_body,
    out_shape=jax.ShapeDtypeStruct((_N, _D), jnp.float32),
)


def _mlp_layer_body(eps_ref, h_ref, a0_ref, a1_ref, W1_ref, b1_ref, g1_ref,
                    be1_ref, W2_ref, b2_ref, g2_ref, be2_ref, o_ref):
    z = (1.0 + eps_ref[...]) * h_ref[...] + (a0_ref[...] + a1_ref[...])
    t = jnp.dot(z, W1_ref[...], preferred_element_type=jnp.float32) + b1_ref[...]
    t = _bn_relu(t, g1_ref[...], be1_ref[...])
    t = jnp.dot(t, W2_ref[...], preferred_element_type=jnp.float32) + b2_ref[...]
    o_ref[...] = _bn_relu(t, g2_ref[...], be2_ref[...])


_mlp_layer = pl.pallas_call(
    _mlp_layer_body,
    out_shape=jax.ShapeDtypeStruct((_N, _D), jnp.float32),
)


def _pool_head_body(batch_ref, h_ref, Wh1_ref, bh1_ref, Wh2_ref, bh2_ref, o_ref):
    gids = lax.broadcasted_iota(jnp.int32, (_G, _N), 0)
    onehot = (batch_ref[...] == gids).astype(jnp.float32)
    sums = jnp.dot(onehot, h_ref[...], preferred_element_type=jnp.float32)
    counts = jnp.sum(onehot, axis=1, keepdims=True)
    pooled = sums / jnp.maximum(counts, 1.0)
    t = jnp.maximum(
        jnp.dot(pooled, Wh1_ref[...], preferred_element_type=jnp.float32)
        + bh1_ref[...], 0.0)
    o_ref[...] = (jnp.dot(t, Wh2_ref[...], preferred_element_type=jnp.float32)
                  + bh2_ref[...])


_pool_head = pl.pallas_call(
    _pool_head_body,
    out_shape=jax.ShapeDtypeStruct((_G, 1), jnp.float32),
)


def kernel(x, edge_index, batch, W_in, b_in, eps, W1, b1, g1, be1,
           W2, b2, g2, be2, Wh1, bh1, Wh2, bh2):
    src = edge_index[0].reshape(_E // _CH, _CH)
    dst = edge_index[1].reshape(_E // _CH, _CH)
    batch2d = batch.reshape(1, _N)

    h = _mlp_in(x, W_in, b_in.reshape(1, _D))
    for l in range(_L):
        parts = _sc_gather_scatter(h, src, dst)
        eps_b = jnp.broadcast_to(eps[l].reshape(1, 1), (1, _D))
        h = _mlp_layer(eps_b, h, parts[0], parts[1],
                       W1[l], b1[l].reshape(1, _D), g1[l].reshape(1, _D),
                       be1[l].reshape(1, _D),
                       W2[l], b2[l].reshape(1, _D), g2[l].reshape(1, _D),
                       be2[l].reshape(1, _D))

    return _pool_head(batch2d, h, Wh1, bh1.reshape(1, _D // 2),
                      Wh2, bh2.reshape(1, 1))
